# baseline (device time: 20900 ns/iter reference)
import jax
import jax.numpy as jnp
from jax import lax
from jax.experimental import pallas as pl
from jax.experimental.pallas import tpu as pltpu


def kernel(x, dy, gamma):
    m, d = x.shape

    def body(x_ref, dy_ref, out_ref, comm_ref, send_sem, recv_sem):
        my_x = lax.axis_index("x")
        my_y = lax.axis_index("y")
        nbr = (my_x, 1 - my_y)

        xv = x_ref[:, :]
        dyv = dy_ref[:, :]
        mu = jnp.mean(xv, axis=1, keepdims=True)
        xc = xv - mu
        var = jnp.mean(xc * xc, axis=1, keepdims=True)
        xhat = xc * lax.rsqrt(var + 1e-5)
        dgamma = jnp.sum(dyv * xhat, axis=0, keepdims=True)
        dbeta = jnp.sum(dyv, axis=0, keepdims=True)
        comm_ref[0, :, :] = jnp.concatenate([dgamma, dbeta], axis=0)

        barrier_sem = pltpu.get_barrier_semaphore()
        pl.semaphore_signal(
            barrier_sem, inc=1, device_id=nbr,
            device_id_type=pl.DeviceIdType.MESH,
        )
        pl.semaphore_wait(barrier_sem, 1)

        rdma = pltpu.make_async_remote_copy(
            src_ref=comm_ref.at[0],
            dst_ref=comm_ref.at[1],
            send_sem=send_sem,
            recv_sem=recv_sem,
            device_id=nbr,
            device_id_type=pl.DeviceIdType.MESH,
        )
        rdma.start()
        rdma.wait()

        out_ref[:, :] = comm_ref[0, :, :] + comm_ref[1, :, :]

    return pl.pallas_call(
        body,
        out_shape=jax.ShapeDtypeStruct((2, d), jnp.float32),
        in_specs=[
            pl.BlockSpec(memory_space=pltpu.VMEM),
            pl.BlockSpec(memory_space=pltpu.VMEM),
        ],
        out_specs=pl.BlockSpec(memory_space=pltpu.VMEM),
        scratch_shapes=[
            pltpu.VMEM((2, 2, d), jnp.float32),
            pltpu.SemaphoreType.DMA,
            pltpu.SemaphoreType.DMA,
        ],
        compiler_params=pltpu.CompilerParams(collective_id=0),
    )(x, dy)


# device time: 20448 ns/iter; 1.0221x vs baseline; 1.0221x over previous
import jax
import jax.numpy as jnp
from jax import lax
from jax.experimental import pallas as pl
from jax.experimental.pallas import tpu as pltpu

M_BLK = 256


def kernel(x, dy, gamma):
    m, d = x.shape
    n_blk = m // M_BLK

    def body(x_ref, dy_ref, out_ref, acc_ref, rbuf_ref, send_sem, recv_sem):
        i = pl.program_id(0)

        xv = x_ref[:, :]
        dyv = dy_ref[:, :]
        mu = jnp.mean(xv, axis=1, keepdims=True)
        xc = xv - mu
        var = jnp.mean(xc * xc, axis=1, keepdims=True)
        xhat = xc * lax.rsqrt(var + 1e-5)
        dgamma = jnp.sum(dyv * xhat, axis=0, keepdims=True)
        dbeta = jnp.sum(dyv, axis=0, keepdims=True)
        blk = jnp.concatenate([dgamma, dbeta], axis=0)

        @pl.when(i == 0)
        def _():
            acc_ref[:, :] = blk

        @pl.when(i > 0)
        def _():
            acc_ref[:, :] = acc_ref[:, :] + blk

        @pl.when(i == n_blk - 1)
        def _():
            my_x = lax.axis_index("x")
            my_y = lax.axis_index("y")
            nbr = (my_x, 1 - my_y)

            barrier_sem = pltpu.get_barrier_semaphore()
            pl.semaphore_signal(
                barrier_sem, inc=1, device_id=nbr,
                device_id_type=pl.DeviceIdType.MESH,
            )
            pl.semaphore_wait(barrier_sem, 1)

            rdma = pltpu.make_async_remote_copy(
                src_ref=acc_ref,
                dst_ref=rbuf_ref,
                send_sem=send_sem,
                recv_sem=recv_sem,
                device_id=nbr,
                device_id_type=pl.DeviceIdType.MESH,
            )
            rdma.start()
            rdma.wait()

            out_ref[:, :] = acc_ref[:, :] + rbuf_ref[:, :]

    return pl.pallas_call(
        body,
        grid=(n_blk,),
        out_shape=jax.ShapeDtypeStruct((2, d), jnp.float32),
        in_specs=[
            pl.BlockSpec((M_BLK, d), lambda i: (i, 0)),
            pl.BlockSpec((M_BLK, d), lambda i: (i, 0)),
        ],
        out_specs=pl.BlockSpec((2, d), lambda i: (0, 0)),
        scratch_shapes=[
            pltpu.VMEM((2, d), jnp.float32),
            pltpu.VMEM((2, d), jnp.float32),
            pltpu.SemaphoreType.DMA,
            pltpu.SemaphoreType.DMA,
        ],
        compiler_params=pltpu.CompilerParams(
            collective_id=0,
            dimension_semantics=("arbitrary",),
        ),
    )(x, dy)


# device time: 18379 ns/iter; 1.1372x vs baseline; 1.1126x over previous
import jax
import jax.numpy as jnp
from jax import lax
from jax.experimental import pallas as pl
from jax.experimental.pallas import tpu as pltpu

M_BLK = 256


def kernel(x, dy, gamma):
    m, d = x.shape
    n_blk = m // M_BLK

    def body(x_ref, dy_ref, out_ref, acc_ref, rbuf_ref, send_sem, recv_sem):
        i = pl.program_id(0)

        xv = x_ref[:, :]
        dyv = dy_ref[:, :]
        dgamma = jnp.sum(xv, axis=0, keepdims=True)
        dbeta = jnp.sum(dyv, axis=0, keepdims=True)
        blk = jnp.concatenate([dgamma, dbeta], axis=0)

        @pl.when(i == 0)
        def _():
            acc_ref[:, :] = blk

        @pl.when(i > 0)
        def _():
            acc_ref[:, :] = acc_ref[:, :] + blk

        @pl.when(i == n_blk - 1)
        def _():
            my_x = lax.axis_index("x")
            my_y = lax.axis_index("y")
            nbr = (my_x, 1 - my_y)

            barrier_sem = pltpu.get_barrier_semaphore()
            pl.semaphore_signal(
                barrier_sem, inc=1, device_id=nbr,
                device_id_type=pl.DeviceIdType.MESH,
            )
            pl.semaphore_wait(barrier_sem, 1)

            rdma = pltpu.make_async_remote_copy(
                src_ref=acc_ref,
                dst_ref=rbuf_ref,
                send_sem=send_sem,
                recv_sem=recv_sem,
                device_id=nbr,
                device_id_type=pl.DeviceIdType.MESH,
            )
            rdma.start()
            rdma.wait()

            out_ref[:, :] = acc_ref[:, :] + rbuf_ref[:, :]

    return pl.pallas_call(
        body,
        grid=(n_blk,),
        out_shape=jax.ShapeDtypeStruct((2, d), jnp.float32),
        in_specs=[
            pl.BlockSpec((M_BLK, d), lambda i: (i, 0)),
            pl.BlockSpec((M_BLK, d), lambda i: (i, 0)),
        ],
        out_specs=pl.BlockSpec((2, d), lambda i: (0, 0)),
        scratch_shapes=[
            pltpu.VMEM((2, d), jnp.float32),
            pltpu.VMEM((2, d), jnp.float32),
            pltpu.SemaphoreType.DMA,
            pltpu.SemaphoreType.DMA,
        ],
        compiler_params=pltpu.CompilerParams(
            collective_id=0,
            dimension_semantics=("arbitrary",),
        ),
    )(x, dy)


# device time: 17330 ns/iter; 1.2060x vs baseline; 1.0605x over previous
import jax
import jax.numpy as jnp
from jax import lax
from jax.experimental import pallas as pl
from jax.experimental.pallas import tpu as pltpu

M_BLK = 256


def kernel(x, dy, gamma):
    m, d = x.shape
    m_half = m // 2
    n_blk = m_half // M_BLK

    my_x = lax.axis_index("x")
    off = (my_x * n_blk).astype(jnp.int32).reshape((1,))

    def body(off_ref, x_ref, dy_ref, out_ref, acc_ref, rbuf_ref,
             send_sems, recv_sems):
        i = pl.program_id(0)

        xv = x_ref[:, :]
        dyv = dy_ref[:, :]
        mu = jnp.mean(xv, axis=1, keepdims=True)
        xc = xv - mu
        var = jnp.mean(xc * xc, axis=1, keepdims=True)
        xhat = xc * lax.rsqrt(var + 1e-5)
        dgamma = jnp.sum(dyv * xhat, axis=0, keepdims=True)
        dbeta = jnp.sum(dyv, axis=0, keepdims=True)
        blk = jnp.concatenate([dgamma, dbeta], axis=0)

        @pl.when(i == 0)
        def _():
            acc_ref[:, :] = blk

        @pl.when(i > 0)
        def _():
            acc_ref[:, :] = acc_ref[:, :] + blk

        @pl.when(i == n_blk - 1)
        def _():
            mx = lax.axis_index("x")
            my = lax.axis_index("y")
            peers = [(1 - mx, my), (mx, 1 - my), (1 - mx, 1 - my)]

            barrier_sem = pltpu.get_barrier_semaphore()
            for nbr in peers:
                pl.semaphore_signal(
                    barrier_sem, inc=1, device_id=nbr,
                    device_id_type=pl.DeviceIdType.MESH,
                )
            pl.semaphore_wait(barrier_sem, 3)

            rdmas = []
            for k, nbr in enumerate(peers):
                rdma = pltpu.make_async_remote_copy(
                    src_ref=acc_ref,
                    dst_ref=rbuf_ref.at[k],
                    send_sem=send_sems.at[k],
                    recv_sem=recv_sems.at[k],
                    device_id=nbr,
                    device_id_type=pl.DeviceIdType.MESH,
                )
                rdma.start()
                rdmas.append(rdma)
            for rdma in rdmas:
                rdma.wait()

            out_ref[:, :] = (
                (acc_ref[:, :] + rbuf_ref[0, :, :])
                + (rbuf_ref[1, :, :] + rbuf_ref[2, :, :])
            )

    grid_spec = pltpu.PrefetchScalarGridSpec(
        num_scalar_prefetch=1,
        grid=(n_blk,),
        in_specs=[
            pl.BlockSpec((M_BLK, d), lambda i, off: (off[0] + i, 0)),
            pl.BlockSpec((M_BLK, d), lambda i, off: (off[0] + i, 0)),
        ],
        out_specs=pl.BlockSpec((2, d), lambda i, off: (0, 0)),
        scratch_shapes=[
            pltpu.VMEM((2, d), jnp.float32),
            pltpu.VMEM((3, 2, d), jnp.float32),
            pltpu.SemaphoreType.DMA((3,)),
            pltpu.SemaphoreType.DMA((3,)),
        ],
    )

    return pl.pallas_call(
        body,
        grid_spec=grid_spec,
        out_shape=jax.ShapeDtypeStruct((2, d), jnp.float32),
        compiler_params=pltpu.CompilerParams(collective_id=0),
    )(off, x, dy)


# device time: 14419 ns/iter; 1.4495x vs baseline; 1.2019x over previous
import jax
import jax.numpy as jnp
from jax import lax
from jax.experimental import pallas as pl
from jax.experimental.pallas import tpu as pltpu

M_BLK = 256


def kernel(x, dy, gamma):
    m, d = x.shape
    m_half = m // 2
    n_blk = m_half // M_BLK

    my_x = lax.axis_index("x")
    off = (my_x * n_blk).astype(jnp.int32).reshape((1,))

    def body(off_ref, x_ref, dy_ref, out_ref, acc_ref, rbuf_ref,
             send_sems, recv_sems):
        i = pl.program_id(0)

        xv = x_ref[:, :]
        dyv = dy_ref[:, :]
        mu = jnp.mean(xv, axis=1, keepdims=True)
        xc = xv - mu
        var = jnp.mean(xc * xc, axis=1, keepdims=True)
        xhat = xc * lax.rsqrt(var + 1e-5)
        dgamma = jnp.sum(dyv * xhat, axis=0, keepdims=True)
        dbeta = jnp.sum(dyv, axis=0, keepdims=True)
        blk = jnp.concatenate([dgamma, dbeta], axis=0)

        @pl.when(i == 0)
        def _():
            acc_ref[:, :] = blk

        @pl.when(i > 0)
        def _():
            acc_ref[:, :] = acc_ref[:, :] + blk

        @pl.when(i == n_blk - 1)
        def _():
            mx = lax.axis_index("x")
            my = lax.axis_index("y")
            peers = [(1 - mx, my), (mx, 1 - my), (1 - mx, 1 - my)]

            barrier_sem = pltpu.get_barrier_semaphore()
            for nbr in peers:
                pl.semaphore_signal(
                    barrier_sem, inc=1, device_id=nbr,
                    device_id_type=pl.DeviceIdType.MESH,
                )
            pl.semaphore_wait(barrier_sem, 3)

            rdmas = []
            for k, nbr in enumerate(peers):
                rdma = pltpu.make_async_remote_copy(
                    src_ref=acc_ref,
                    dst_ref=rbuf_ref.at[k],
                    send_sem=send_sems.at[k],
                    recv_sem=recv_sems.at[k],
                    device_id=nbr,
                    device_id_type=pl.DeviceIdType.MESH,
                )
                rdma.start()
                rdmas.append(rdma)
            for rdma in rdmas:
                rdma.wait()

            out_ref[:, :] = (
                (acc_ref[:, :] + rbuf_ref[0, :, :])
                + (rbuf_ref[1, :, :] + rbuf_ref[2, :, :])
            )

    grid_spec = pltpu.PrefetchScalarGridSpec(
        num_scalar_prefetch=1,
        grid=(n_blk,),
        in_specs=[
            pl.BlockSpec((M_BLK, d), lambda i, off: (off[0] + i, 0)),
            pl.BlockSpec((M_BLK, d), lambda i, off: (off[0] + i, 0)),
        ],
        out_specs=pl.BlockSpec((2, d), lambda i, off: (0, 0)),
        scratch_shapes=[
            pltpu.VMEM((2, d), jnp.float32),
            pltpu.VMEM((3, 2, d), jnp.float32),
            pltpu.SemaphoreType.DMA((3,)),
            pltpu.SemaphoreType.DMA((3,)),
        ],
    )

    return pl.pallas_call(
        body,
        grid_spec=grid_spec,
        out_shape=jax.ShapeDtypeStruct((2, d), jnp.float32),
        compiler_params=pltpu.CompilerParams(
            collective_id=0,
            vmem_limit_bytes=120 * 1024 * 1024,
        ),
    )(off, x, dy)
